# Initial kernel scaffold; baseline (speedup 1.0000x reference)
import jax
import jax.numpy as jnp
from jax.experimental import pallas as pl


def kernel(x, pos, edge_index, batch, W_in1, b_in1, W_out1, b_out1, W_in2, b_in2, W_out2, b_out2, W_in3, b_in3, W_out3, b_out3, W_lin, b_lin):
    raise NotImplementedError("write your pallas kernel here")


# speedup vs baseline: 1.7630x; 1.7630x over previous
import functools

import jax
import jax.numpy as jnp
from jax import lax
from jax.experimental import pallas as pl
from jax.experimental.pallas import tpu as pltpu
from jax.experimental.pallas import tpu_sc as plsc

N = 100000
E = 3200000
D = 16
H = 64
G = 64
NCLS = 10

NCORES = 2
NSUB = 16
LANES = 16

NRANGE = 8
NPASS = NRANGE // NCORES
RNG = N // NRANGE
RPAD = 12800
EPT = E // NSUB
BLK = 800
NBLK = EPT // BLK
NGRP = BLK // LANES
ROWS_PT = RPAD // NSUB


def _sc_layer_body(h_hbm, px_hbm, py_hbm, src_hbm, dst_hbm, w_hbm, b_hbm,
                   out_hbm,
                   agg_sh, src_v, dst_v, dstl_v, psx_v, psy_v, pdx_v, pdy_v,
                   xrows_v, msg_v, w_v, b_v, sem):
    c = lax.axis_index("c")
    s = lax.axis_index("s")
    pltpu.sync_copy(w_hbm, w_v)
    pltpu.sync_copy(b_hbm, b_v)
    wa = [w_v[pl.ds(16 * k, 16)] for k in range(4)]
    wb = [w_v[pl.ds(H + 16 * k, 16)] for k in range(4)]
    bb = [b_v[pl.ds(16 * k, 16)] for k in range(4)]
    iota = lax.iota(jnp.int32, 16)

    for p in range(NPASS):
        r = NPASS * c + p
        base = r * RPAD

        def zloop(e, carry):
            for k in range(4):
                msg_v[e, pl.ds(16 * k, 16)] = jnp.zeros((16,), jnp.float32)
            return carry
        lax.fori_loop(0, BLK, zloop, 0)
        for q in range(ROWS_PT // BLK):
            pltpu.sync_copy(msg_v, agg_sh.at[pl.ds(s * ROWS_PT + q * BLK, BLK)])
        plsc.subcore_barrier()

        def bloop(bi, carry):
            eoff = s * EPT + bi * BLK
            pltpu.sync_copy(src_hbm.at[pl.ds(eoff, BLK)], src_v)
            pltpu.sync_copy(dst_hbm.at[pl.ds(eoff, BLK)], dst_v)
            pltpu.async_copy(px_hbm.at[src_v], psx_v, sem).wait()
            pltpu.async_copy(py_hbm.at[src_v], psy_v, sem).wait()
            pltpu.async_copy(px_hbm.at[dst_v], pdx_v, sem).wait()
            pltpu.async_copy(py_hbm.at[dst_v], pdy_v, sem).wait()
            pltpu.async_copy(h_hbm.at[src_v], xrows_v, sem).wait()

            def gloop(g, gc):
                gb = g * LANES
                d16 = dst_v[pl.ds(gb, LANES)]
                m = (d16 >= base) & (d16 < base + RNG)
                dump = RNG + (g % 18) * 16 + iota
                dstl_v[pl.ds(gb, LANES)] = jnp.where(m, d16 - base, dump)
                u16 = psx_v[pl.ds(gb, LANES)] - pdx_v[pl.ds(gb, LANES)]
                v16 = psy_v[pl.ds(gb, LANES)] - pdy_v[pl.ds(gb, LANES)]
                for j in range(LANES):
                    e = gb + j
                    jj = jnp.full((LANES,), j, jnp.int32)
                    us = jnp.take_along_axis(u16, jj, axis=0,
                                             mode="promise_in_bounds")
                    vs = jnp.take_along_axis(v16, jj, axis=0,
                                             mode="promise_in_bounds")
                    xr = xrows_v[e, :]
                    for k in range(4):
                        t = us * wa[k] + vs * wb[k] + bb[k]
                        t = jnp.maximum(t, 0.0)
                        msg_v[e, pl.ds(16 * k, 16)] = t * xr
                return gc
            lax.fori_loop(0, NGRP, gloop, 0)
            pltpu.sync_copy(msg_v, agg_sh.at[dstl_v], add=True)
            return carry
        lax.fori_loop(0, NBLK, bloop, 0)
        plsc.subcore_barrier()

        pltpu.sync_copy(agg_sh.at[pl.ds(s * ROWS_PT, ROWS_PT)],
                        out_hbm.at[r, pl.ds(s * ROWS_PT, ROWS_PT)])
        plsc.subcore_barrier()


def _sc_layer(h, pos_x, pos_y, src, dst, w_flat, b_in):
    mesh = plsc.VectorSubcoreMesh(core_axis_name="c", subcore_axis_name="s",
                                  num_cores=NCORES, num_subcores=NSUB)
    kfn = pl.kernel(
        _sc_layer_body,
        out_type=jax.ShapeDtypeStruct((NRANGE, RPAD, H), jnp.float32),
        mesh=mesh,
        compiler_params=pltpu.CompilerParams(use_tc_tiling_on_sc=False),
        scratch_types=[
            pltpu.VMEM_SHARED((RPAD, H), jnp.float32),
            pltpu.VMEM((BLK,), jnp.int32),
            pltpu.VMEM((BLK,), jnp.int32),
            pltpu.VMEM((BLK,), jnp.int32),
            pltpu.VMEM((BLK,), jnp.float32),
            pltpu.VMEM((BLK,), jnp.float32),
            pltpu.VMEM((BLK,), jnp.float32),
            pltpu.VMEM((BLK,), jnp.float32),
            pltpu.VMEM((BLK, D), jnp.float32),
            pltpu.VMEM((BLK, H), jnp.float32),
            pltpu.VMEM((2 * H,), jnp.float32),
            pltpu.VMEM((H,), jnp.float32),
            pltpu.SemaphoreType.DMA,
        ],
    )
    return kfn(h, pos_x, pos_y, src, dst, w_flat, b_in)


NPAD = NRANGE * RPAD


def _tc_affine(agg2, w_out, b_out2):
    AB = 512

    def body(agg_ref, w_ref, b_ref, out_ref):
        out_ref[...] = (
            jnp.dot(agg_ref[...], w_ref[...],
                    preferred_element_type=jnp.float32,
                    precision=lax.Precision.HIGHEST)
            + b_ref[...])

    return pl.pallas_call(
        body,
        grid=(NPAD // AB,),
        in_specs=[
            pl.BlockSpec((AB, H), lambda i: (i, 0)),
            pl.BlockSpec((H, D), lambda i: (0, 0)),
            pl.BlockSpec((1, D), lambda i: (0, 0)),
        ],
        out_specs=pl.BlockSpec((AB, D), lambda i: (i, 0)),
        out_shape=jax.ShapeDtypeStruct((NPAD, D), jnp.float32),
    )(agg2, w_out, b_out2)


def _tc_head(h, batch3, w_lin, b_lin2):
    BN = 1000
    NB = N // BN

    def body(h_ref, b_ref, wl_ref, bl_ref, out_ref, acc, cnt):
        i = pl.program_id(0)

        @pl.when(i == 0)
        def _init():
            acc[...] = jnp.zeros_like(acc)
            cnt[...] = jnp.zeros_like(cnt)

        gi = lax.broadcasted_iota(jnp.int32, (G, BN), 0)
        oht = (b_ref[0] == gi).astype(jnp.float32)
        acc[...] += lax.dot_general(oht, h_ref[...],
                                    (((1,), (0,)), ((), ())),
                                    preferred_element_type=jnp.float32,
                                    precision=lax.Precision.HIGHEST)
        cnt[...] += jnp.sum(oht, axis=1, keepdims=True)

        @pl.when(i == NB - 1)
        def _fin():
            pooled = acc[...] / jnp.maximum(cnt[...], 1.0)
            logits = (jnp.dot(pooled, wl_ref[...],
                              preferred_element_type=jnp.float32,
                              precision=lax.Precision.HIGHEST)
                      + bl_ref[...])
            mx = jnp.max(logits, axis=0, keepdims=True)
            z = logits - mx
            lse = jnp.log(jnp.sum(jnp.exp(z), axis=0, keepdims=True))
            out_ref[...] = z - lse

    return pl.pallas_call(
        body,
        grid=(NB,),
        in_specs=[
            pl.BlockSpec((BN, D), lambda i: (i, 0)),
            pl.BlockSpec((1, 1, BN), lambda i: (i, 0, 0)),
            pl.BlockSpec((D, NCLS), lambda i: (0, 0)),
            pl.BlockSpec((1, NCLS), lambda i: (0, 0)),
        ],
        out_specs=pl.BlockSpec((G, NCLS), lambda i: (0, 0)),
        out_shape=jax.ShapeDtypeStruct((G, NCLS), jnp.float32),
        scratch_shapes=[pltpu.VMEM((G, D), jnp.float32),
                        pltpu.VMEM((G, 1), jnp.float32)],
    )(h, batch3, w_lin, b_lin2)


def kernel(x, pos, edge_index, batch,
           W_in1, b_in1, W_out1, b_out1,
           W_in2, b_in2, W_out2, b_out2,
           W_in3, b_in3, W_out3, b_out3,
           W_lin, b_lin):
    pad_w = ((0, 0), (0, RPAD - RNG), (0, 0))
    srcp = edge_index[0] + (edge_index[0] // RNG) * (RPAD - RNG)
    dstp = edge_index[1] + (edge_index[1] // RNG) * (RPAD - RNG)
    pos_x = jnp.pad(pos[:, 0].reshape(NRANGE, RNG, 1), pad_w).reshape(NPAD)
    pos_y = jnp.pad(pos[:, 1].reshape(NRANGE, RNG, 1), pad_w).reshape(NPAD)
    x_pad = jnp.pad(x.reshape(NRANGE, RNG, D), pad_w).reshape(NPAD, D)
    batch3 = batch.reshape(N // 1000, 1, 1000)

    hp = x_pad
    for (wi, bi, wo, bo) in ((W_in1, b_in1, W_out1, b_out1),
                             (W_in2, b_in2, W_out2, b_out2),
                             (W_in3, b_in3, W_out3, b_out3)):
        agg = _sc_layer(hp, pos_x, pos_y, srcp, dstp, wi.reshape(2 * H), bi)
        hp = _tc_affine(agg.reshape(NPAD, H), wo, bo.reshape(1, D))
    h = hp.reshape(NRANGE, RPAD, D)[:, :RNG, :].reshape(N, D)
    return _tc_head(h, batch3, W_lin, b_lin.reshape(1, NCLS))


# speedup vs baseline: 4.5006x; 2.5528x over previous
import jax
import jax.numpy as jnp
from jax import lax
from jax.experimental import pallas as pl
from jax.experimental.pallas import tpu as pltpu
from jax.experimental.pallas import tpu_sc as plsc

N = 100000
E = 3200000
D = 16
H = 64
G = 64
NCLS = 10

NCORES = 2
NSUB = 16
LANES = 16

NRANGE = 8
NPASS = NRANGE // NCORES
RNG = N // NRANGE
RPAD = 16384
RSH = 14
NPAD = NRANGE * RPAD
EPT = E // NSUB
EPC = EPT // NCORES

BLK = 512
NGRP = BLK // LANES
CAP = (EPC // BLK + 1) * BLK + BLK
ROWS_PT = RPAD // NSUB

BI = 2000
NBI = EPC // BI
NGI = BI // LANES


def _lane_prefix_incl(v, iota):
    for dlt in (1, 2, 4, 8):
        idx = jnp.maximum(iota - dlt, 0)
        sh = jnp.take_along_axis(v, idx, axis=0, mode="promise_in_bounds")
        v = v + jnp.where(iota >= dlt, sh, 0)
    return v


def _bin_body(px_hbm, py_hbm, src_hbm, dst_hbm,
              srcl_hbm, dstl_hbm, ul_hbm, vl_hbm, tot_hbm,
              srcb, dstb, slot_v, dstl_v, psx, psy, pdx, pdy,
              u_v, v_v, tv, sem):
    c = lax.axis_index("c")
    s = lax.axis_index("s")
    iota = lax.iota(jnp.int32, LANES)
    eoff0 = s * EPT + c * EPC

    def abloop(bi, carry):
        pltpu.sync_copy(src_hbm.at[pl.ds(eoff0 + bi * BI, BI)], srcb)
        pltpu.sync_copy(dst_hbm.at[pl.ds(eoff0 + bi * BI, BI)], dstb)

        def agloop(g, cnts):
            d16 = dstb[pl.ds(g * LANES, LANES)]
            r16 = lax.shift_right_logical(d16, RSH)
            return tuple(cnts[r] + jnp.where(r16 == r, 1, 0)
                         for r in range(NRANGE))
        return lax.fori_loop(0, NGI, agloop, carry)

    zero16 = jnp.zeros((LANES,), jnp.int32)
    cnts = lax.fori_loop(0, NBI, abloop, (zero16,) * NRANGE)

    base_vecs = []
    for r in range(NRANGE):
        flatbase = ((c * NSUB + s) * NRANGE + r) * CAP
        incl = _lane_prefix_incl(cnts[r], iota)
        base_vecs.append(flatbase + incl - cnts[r])
        tot = jnp.take_along_axis(incl, jnp.full((LANES,), 15, jnp.int32),
                                  axis=0, mode="promise_in_bounds")
        tv[...] = tot
        pltpu.sync_copy(tv, tot_hbm.at[c, s, r])

    def bbloop(bi, carry):
        pltpu.sync_copy(src_hbm.at[pl.ds(eoff0 + bi * BI, BI)], srcb)
        pltpu.sync_copy(dst_hbm.at[pl.ds(eoff0 + bi * BI, BI)], dstb)
        d1 = pltpu.async_copy(px_hbm.at[srcb], psx, sem)
        d2 = pltpu.async_copy(py_hbm.at[srcb], psy, sem)
        d3 = pltpu.async_copy(px_hbm.at[dstb], pdx, sem)
        d4 = pltpu.async_copy(py_hbm.at[dstb], pdy, sem)
        d1.wait(); d2.wait(); d3.wait(); d4.wait()

        def bgloop(g, cnts):
            gb = g * LANES
            d16 = dstb[pl.ds(gb, LANES)]
            r16 = lax.shift_right_logical(d16, RSH)
            dstl_v[pl.ds(gb, LANES)] = d16 & (RPAD - 1)
            u_v[pl.ds(gb, LANES)] = psx[pl.ds(gb, LANES)] - pdx[pl.ds(gb, LANES)]
            v_v[pl.ds(gb, LANES)] = psy[pl.ds(gb, LANES)] - pdy[pl.ds(gb, LANES)]
            slot = zero16
            ncnts = []
            for r in range(NRANGE):
                mr = r16 == r
                slot = jnp.where(mr, base_vecs[r] + cnts[r], slot)
                ncnts.append(cnts[r] + jnp.where(mr, 1, 0))
            slot_v[pl.ds(gb, LANES)] = slot
            return tuple(ncnts)
        carry = lax.fori_loop(0, NGI, bgloop, carry)
        e1 = pltpu.async_copy(srcb, srcl_hbm.at[slot_v], sem)
        e2 = pltpu.async_copy(dstl_v, dstl_hbm.at[slot_v], sem)
        e3 = pltpu.async_copy(u_v, ul_hbm.at[slot_v], sem)
        e4 = pltpu.async_copy(v_v, vl_hbm.at[slot_v], sem)
        e1.wait(); e2.wait(); e3.wait(); e4.wait()
        return carry

    lax.fori_loop(0, NBI, bbloop, (zero16,) * NRANGE)


def _bin_edges(pos_x, pos_y, srcp, dstp):
    mesh = plsc.VectorSubcoreMesh(core_axis_name="c", subcore_axis_name="s",
                                  num_cores=NCORES, num_subcores=NSUB)
    nlist = NCORES * NSUB * NRANGE * CAP
    kfn = pl.kernel(
        _bin_body,
        out_type=(
            jax.ShapeDtypeStruct((nlist,), jnp.int32),
            jax.ShapeDtypeStruct((nlist,), jnp.int32),
            jax.ShapeDtypeStruct((nlist,), jnp.float32),
            jax.ShapeDtypeStruct((nlist,), jnp.float32),
            jax.ShapeDtypeStruct((NCORES, NSUB, NRANGE, LANES), jnp.int32),
        ),
        mesh=mesh,
        compiler_params=pltpu.CompilerParams(use_tc_tiling_on_sc=False),
        scratch_types=[
            pltpu.VMEM((BI,), jnp.int32),
            pltpu.VMEM((BI,), jnp.int32),
            pltpu.VMEM((BI,), jnp.int32),
            pltpu.VMEM((BI,), jnp.int32),
            pltpu.VMEM((BI,), jnp.float32),
            pltpu.VMEM((BI,), jnp.float32),
            pltpu.VMEM((BI,), jnp.float32),
            pltpu.VMEM((BI,), jnp.float32),
            pltpu.VMEM((BI,), jnp.float32),
            pltpu.VMEM((BI,), jnp.float32),
            pltpu.VMEM((LANES,), jnp.int32),
            pltpu.SemaphoreType.DMA,
        ],
    )
    return kfn(pos_x, pos_y, srcp, dstp)


def _sc_layer_body(h_hbm, srcl_hbm, dstl_hbm, ul_hbm, vl_hbm, tot_hbm,
                   w_hbm, b_hbm, out_hbm,
                   agg_sh, src_v, dstl_v, u_v, v_v, xrows_v, msg_v,
                   w_v, b_v, tv, sem):
    c = lax.axis_index("c")
    s = lax.axis_index("s")
    iota = lax.iota(jnp.int32, LANES)
    pltpu.sync_copy(w_hbm, w_v)
    pltpu.sync_copy(b_hbm, b_v)
    wa = [w_v[pl.ds(16 * k, 16)] for k in range(4)]
    wb = [w_v[pl.ds(H + 16 * k, 16)] for k in range(4)]
    bb = [b_v[pl.ds(16 * k, 16)] for k in range(4)]

    for p in range(NPASS):
        r = NPASS * c + p

        def zloop(e, carry):
            for k in range(4):
                msg_v[e, pl.ds(16 * k, 16)] = jnp.zeros((16,), jnp.float32)
            return carry
        lax.fori_loop(0, BLK, zloop, 0)
        for q in range(ROWS_PT // BLK):
            pltpu.sync_copy(msg_v, agg_sh.at[pl.ds(s * ROWS_PT + q * BLK, BLK)])
        plsc.subcore_barrier()

        for cc in range(NCORES):
            flatbase = ((cc * NSUB + s) * NRANGE + r) * CAP
            pltpu.sync_copy(tot_hbm.at[cc, s, r], tv)
            tvec = tv[...]
            total = tvec[0]
            trips = lax.shift_right_logical(total + (BLK - 1), 9)

            def bloop(bi, carry, flatbase=flatbase, tvec=tvec, total=total):
                o = flatbase + bi * BLK
                pltpu.sync_copy(srcl_hbm.at[pl.ds(o, BLK)], src_v)
                pltpu.sync_copy(dstl_hbm.at[pl.ds(o, BLK)], dstl_v)

                def fix_tail():
                    def floop(g, fc):
                        gb = g * LANES
                        lane = bi * BLK + gb + iota
                        m = lane < tvec
                        cs = src_v[pl.ds(gb, LANES)]
                        cd = dstl_v[pl.ds(gb, LANES)]
                        src_v[pl.ds(gb, LANES)] = jnp.where(m, cs, 0)
                        dstl_v[pl.ds(gb, LANES)] = jnp.where(
                            m, cd, RNG + (g % 64) * 16 + iota)
                        return fc
                    lax.fori_loop(0, NGRP, floop, 0)
                lax.cond(bi * BLK + BLK > total, fix_tail, lambda: None)

                d1 = pltpu.async_copy(h_hbm.at[src_v], xrows_v, sem)
                d2 = pltpu.async_copy(ul_hbm.at[pl.ds(o, BLK)], u_v, sem)
                d3 = pltpu.async_copy(vl_hbm.at[pl.ds(o, BLK)], v_v, sem)
                d1.wait(); d2.wait(); d3.wait()

                def gloop(g, gc):
                    gb = g * LANES
                    u16 = u_v[pl.ds(gb, LANES)]
                    v16 = v_v[pl.ds(gb, LANES)]
                    for j in range(LANES):
                        e = gb + j
                        jj = jnp.full((LANES,), j, jnp.int32)
                        us = jnp.take_along_axis(u16, jj, axis=0,
                                                 mode="promise_in_bounds")
                        vs = jnp.take_along_axis(v16, jj, axis=0,
                                                 mode="promise_in_bounds")
                        xr = xrows_v[e, :]
                        for k in range(4):
                            t = us * wa[k] + vs * wb[k] + bb[k]
                            t = jnp.maximum(t, 0.0)
                            msg_v[e, pl.ds(16 * k, 16)] = t * xr
                    return gc
                lax.fori_loop(0, NGRP, gloop, 0)
                pltpu.sync_copy(msg_v, agg_sh.at[dstl_v], add=True)
                return carry
            lax.fori_loop(0, trips, bloop, 0)
        plsc.subcore_barrier()

        pltpu.sync_copy(agg_sh.at[pl.ds(s * ROWS_PT, ROWS_PT)],
                        out_hbm.at[r, pl.ds(s * ROWS_PT, ROWS_PT)])
        plsc.subcore_barrier()


def _sc_layer(h, edge_lists, w_flat, b_in):
    srcl, dstl, ul, vl, tot = edge_lists
    mesh = plsc.VectorSubcoreMesh(core_axis_name="c", subcore_axis_name="s",
                                  num_cores=NCORES, num_subcores=NSUB)
    kfn = pl.kernel(
        _sc_layer_body,
        out_type=jax.ShapeDtypeStruct((NRANGE, RPAD, H), jnp.float32),
        mesh=mesh,
        compiler_params=pltpu.CompilerParams(use_tc_tiling_on_sc=False),
        scratch_types=[
            pltpu.VMEM_SHARED((RPAD, H), jnp.float32),
            pltpu.VMEM((BLK,), jnp.int32),
            pltpu.VMEM((BLK,), jnp.int32),
            pltpu.VMEM((BLK,), jnp.float32),
            pltpu.VMEM((BLK,), jnp.float32),
            pltpu.VMEM((BLK, D), jnp.float32),
            pltpu.VMEM((BLK, H), jnp.float32),
            pltpu.VMEM((2 * H,), jnp.float32),
            pltpu.VMEM((H,), jnp.float32),
            pltpu.VMEM((LANES,), jnp.int32),
            pltpu.SemaphoreType.DMA,
        ],
    )
    return kfn(h, srcl, dstl, ul, vl, tot, w_flat, b_in)


def _tc_affine(agg2, w_out, b_out2):
    AB = 512

    def body(agg_ref, w_ref, b_ref, out_ref):
        out_ref[...] = (
            jnp.dot(agg_ref[...], w_ref[...],
                    preferred_element_type=jnp.float32,
                    precision=lax.Precision.HIGHEST)
            + b_ref[...])

    return pl.pallas_call(
        body,
        grid=(NPAD // AB,),
        in_specs=[
            pl.BlockSpec((AB, H), lambda i: (i, 0)),
            pl.BlockSpec((H, D), lambda i: (0, 0)),
            pl.BlockSpec((1, D), lambda i: (0, 0)),
        ],
        out_specs=pl.BlockSpec((AB, D), lambda i: (i, 0)),
        out_shape=jax.ShapeDtypeStruct((NPAD, D), jnp.float32),
    )(agg2, w_out, b_out2)


def _tc_head(h, batch3, w_lin, b_lin2):
    BN = 1000
    NB = N // BN

    def body(h_ref, b_ref, wl_ref, bl_ref, out_ref, acc, cnt):
        i = pl.program_id(0)

        @pl.when(i == 0)
        def _init():
            acc[...] = jnp.zeros_like(acc)
            cnt[...] = jnp.zeros_like(cnt)

        gi = lax.broadcasted_iota(jnp.int32, (G, BN), 0)
        oht = (b_ref[0] == gi).astype(jnp.float32)
        acc[...] += lax.dot_general(oht, h_ref[...],
                                    (((1,), (0,)), ((), ())),
                                    preferred_element_type=jnp.float32,
                                    precision=lax.Precision.HIGHEST)
        cnt[...] += jnp.sum(oht, axis=1, keepdims=True)

        @pl.when(i == NB - 1)
        def _fin():
            pooled = acc[...] / jnp.maximum(cnt[...], 1.0)
            logits = (jnp.dot(pooled, wl_ref[...],
                              preferred_element_type=jnp.float32,
                              precision=lax.Precision.HIGHEST)
                      + bl_ref[...])
            mx = jnp.max(logits, axis=0, keepdims=True)
            z = logits - mx
            lse = jnp.log(jnp.sum(jnp.exp(z), axis=0, keepdims=True))
            out_ref[...] = z - lse

    return pl.pallas_call(
        body,
        grid=(NB,),
        in_specs=[
            pl.BlockSpec((BN, D), lambda i: (i, 0)),
            pl.BlockSpec((1, 1, BN), lambda i: (i, 0, 0)),
            pl.BlockSpec((D, NCLS), lambda i: (0, 0)),
            pl.BlockSpec((1, NCLS), lambda i: (0, 0)),
        ],
        out_specs=pl.BlockSpec((G, NCLS), lambda i: (0, 0)),
        out_shape=jax.ShapeDtypeStruct((G, NCLS), jnp.float32),
        scratch_shapes=[pltpu.VMEM((G, D), jnp.float32),
                        pltpu.VMEM((G, 1), jnp.float32)],
    )(h, batch3, w_lin, b_lin2)


def kernel(x, pos, edge_index, batch,
           W_in1, b_in1, W_out1, b_out1,
           W_in2, b_in2, W_out2, b_out2,
           W_in3, b_in3, W_out3, b_out3,
           W_lin, b_lin):
    pad_w = ((0, 0), (0, RPAD - RNG), (0, 0))
    srcp = edge_index[0] + (edge_index[0] // RNG) * (RPAD - RNG)
    dstp = edge_index[1] + (edge_index[1] // RNG) * (RPAD - RNG)
    pos_x = jnp.pad(pos[:, 0].reshape(NRANGE, RNG, 1), pad_w).reshape(NPAD)
    pos_y = jnp.pad(pos[:, 1].reshape(NRANGE, RNG, 1), pad_w).reshape(NPAD)
    x_pad = jnp.pad(x.reshape(NRANGE, RNG, D), pad_w).reshape(NPAD, D)
    batch3 = batch.reshape(N // 1000, 1, 1000)

    edge_lists = _bin_edges(pos_x, pos_y, srcp, dstp)

    hp = x_pad
    for (wi, bi, wo, bo) in ((W_in1, b_in1, W_out1, b_out1),
                             (W_in2, b_in2, W_out2, b_out2),
                             (W_in3, b_in3, W_out3, b_out3)):
        agg = _sc_layer(hp, edge_lists, wi.reshape(2 * H), bi)
        hp = _tc_affine(agg.reshape(NPAD, H), wo, bo.reshape(1, D))
    h = hp.reshape(NRANGE, RPAD, D)[:, :RNG, :].reshape(N, D)
    return _tc_head(h, batch3, W_lin, b_lin.reshape(1, NCLS))


# speedup vs baseline: 7.2450x; 1.6098x over previous
import jax
import jax.numpy as jnp
from jax import lax
from jax.experimental import pallas as pl
from jax.experimental.pallas import tpu as pltpu
from jax.experimental.pallas import tpu_sc as plsc

N = 100000
E = 3200000
D = 16
H = 64
G = 64
NCLS = 10

NCORES = 2
NSUB = 16
LANES = 16

NRANGE = 8
NPASS = NRANGE // NCORES
RNG = N // NRANGE
RPAD = 16384
RSH = 14
NPAD = NRANGE * RPAD
EPT = E // NSUB
EPC = EPT // NCORES

BLK = 512
NGRP = BLK // LANES
CAP = (EPC // BLK + 1) * BLK + BLK
ROWS_PT = RPAD // NSUB

BI = 2000
NBI = EPC // BI
NGI = BI // LANES
PW = 16


def _lane_prefix_incl(v, iota):
    for dlt in (1, 2, 4, 8):
        idx = jnp.maximum(iota - dlt, 0)
        sh = jnp.take_along_axis(v, idx, axis=0, mode="promise_in_bounds")
        v = v + jnp.where(iota >= dlt, sh, 0)
    return v


def _bin_body(pxy_hbm, src_hbm, dst_hbm,
              sdl_hbm, prl_hbm, tot_hbm,
              srcb, dstb, slot_v, sd_v, psr, pdr, prec, tv, sem):
    c = lax.axis_index("c")
    s = lax.axis_index("s")
    iota = lax.iota(jnp.int32, LANES)
    eoff0 = s * EPT + c * EPC

    def abloop(bi, carry):
        pltpu.sync_copy(dst_hbm.at[pl.ds(eoff0 + bi * BI, BI)], dstb)

        def agloop(g, cnts):
            d16 = dstb[pl.ds(g * LANES, LANES)]
            r16 = lax.shift_right_logical(d16, RSH)
            return tuple(cnts[r] + jnp.where(r16 == r, 1, 0)
                         for r in range(NRANGE))
        return lax.fori_loop(0, NGI, agloop, carry)

    zero16 = jnp.zeros((LANES,), jnp.int32)
    cnts = lax.fori_loop(0, NBI, abloop, (zero16,) * NRANGE)

    base_vecs = []
    for r in range(NRANGE):
        flatbase = ((c * NSUB + s) * NRANGE + r) * CAP
        incl = _lane_prefix_incl(cnts[r], iota)
        base_vecs.append(flatbase + incl - cnts[r])
        tot = jnp.take_along_axis(incl, jnp.full((LANES,), 15, jnp.int32),
                                  axis=0, mode="promise_in_bounds")
        tv[...] = tot
        pltpu.sync_copy(tv, tot_hbm.at[c, s, r])

    rot2 = (iota - 2) & (LANES - 1)
    msel = iota < 2

    def bbloop(bi, carry):
        pltpu.sync_copy(src_hbm.at[pl.ds(eoff0 + bi * BI, BI)], srcb)
        pltpu.sync_copy(dst_hbm.at[pl.ds(eoff0 + bi * BI, BI)], dstb)
        d1 = pltpu.async_copy(pxy_hbm.at[srcb], psr, sem)
        d2 = pltpu.async_copy(pxy_hbm.at[dstb], pdr, sem)
        d1.wait(); d2.wait()

        def bgloop(g, cnts):
            gb = g * LANES
            s16 = srcb[pl.ds(gb, LANES)]
            d16 = dstb[pl.ds(gb, LANES)]
            r16 = lax.shift_right_logical(d16, RSH)
            sd_v[pl.ds(gb, LANES)] = (
                lax.shift_left(s16, RSH) | (d16 & (RPAD - 1)))
            slot = zero16
            ncnts = []
            for r in range(NRANGE):
                mr = r16 == r
                slot = jnp.where(mr, base_vecs[r] + cnts[r], slot)
                ncnts.append(cnts[r] + jnp.where(mr, 1, 0))
            slot_v[pl.ds(gb, LANES)] = slot
            for j in range(LANES):
                e = gb + j
                sp = psr[e, :]
                dp = pdr[e, :]
                dp2 = jnp.take_along_axis(dp, rot2, axis=0,
                                          mode="promise_in_bounds")
                prec[e, :] = jnp.where(msel, sp, dp2)
            return tuple(ncnts)
        carry = lax.fori_loop(0, NGI, bgloop, carry)
        e1 = pltpu.async_copy(sd_v, sdl_hbm.at[slot_v], sem)
        e2 = pltpu.async_copy(prec, prl_hbm.at[slot_v], sem)
        e1.wait(); e2.wait()
        return carry

    lax.fori_loop(0, NBI, bbloop, (zero16,) * NRANGE)


def _bin_edges(pos_xy8, srcp, dstp):
    mesh = plsc.VectorSubcoreMesh(core_axis_name="c", subcore_axis_name="s",
                                  num_cores=NCORES, num_subcores=NSUB)
    nlist = NCORES * NSUB * NRANGE * CAP
    kfn = pl.kernel(
        _bin_body,
        out_type=(
            jax.ShapeDtypeStruct((nlist,), jnp.int32),
            jax.ShapeDtypeStruct((nlist, PW), jnp.float32),
            jax.ShapeDtypeStruct((NCORES, NSUB, NRANGE, LANES), jnp.int32),
        ),
        mesh=mesh,
        compiler_params=pltpu.CompilerParams(use_tc_tiling_on_sc=False),
        scratch_types=[
            pltpu.VMEM((BI,), jnp.int32),
            pltpu.VMEM((BI,), jnp.int32),
            pltpu.VMEM((BI,), jnp.int32),
            pltpu.VMEM((BI,), jnp.int32),
            pltpu.VMEM((BI, PW), jnp.float32),
            pltpu.VMEM((BI, PW), jnp.float32),
            pltpu.VMEM((BI, PW), jnp.float32),
            pltpu.VMEM((LANES,), jnp.int32),
            pltpu.SemaphoreType.DMA,
        ],
    )
    return kfn(pos_xy8, srcp, dstp)


def _sc_layer_body(h_hbm, sdl_hbm, prl_hbm, tot_hbm,
                   w_hbm, b_hbm, out_hbm,
                   agg_sh, src_v, dstl_v, prec_v, xrows_v, msg_v,
                   w_v, b_v, tv, sem):
    c = lax.axis_index("c")
    s = lax.axis_index("s")
    iota = lax.iota(jnp.int32, LANES)
    pltpu.sync_copy(w_hbm, w_v)
    pltpu.sync_copy(b_hbm, b_v)
    wa = [w_v[pl.ds(16 * k, 16)] for k in range(4)]
    wb = [w_v[pl.ds(H + 16 * k, 16)] for k in range(4)]
    bb = [b_v[pl.ds(16 * k, 16)] for k in range(4)]
    z0 = jnp.full((LANES,), 0, jnp.int32)
    z2 = jnp.full((LANES,), 2, jnp.int32)
    o1 = jnp.full((LANES,), 1, jnp.int32)
    o3 = jnp.full((LANES,), 3, jnp.int32)

    for p in range(NPASS):
        r = NPASS * c + p

        def zloop(e, carry):
            for k in range(4):
                msg_v[e, pl.ds(16 * k, 16)] = jnp.zeros((16,), jnp.float32)
            return carry
        lax.fori_loop(0, BLK, zloop, 0)
        for q in range(ROWS_PT // BLK):
            pltpu.sync_copy(msg_v, agg_sh.at[pl.ds(s * ROWS_PT + q * BLK, BLK)])
        plsc.subcore_barrier()

        for cc in range(NCORES):
            flatbase = ((cc * NSUB + s) * NRANGE + r) * CAP
            pltpu.sync_copy(tot_hbm.at[cc, s, r], tv)
            tvec = tv[...]
            total = tvec[0]
            trips = lax.shift_right_logical(total + (BLK - 1), 9)

            def bloop(bi, carry, flatbase=flatbase, tvec=tvec, total=total):
                o = flatbase + bi * BLK
                pltpu.sync_copy(sdl_hbm.at[pl.ds(o, BLK)], src_v)
                d1 = pltpu.async_copy(prl_hbm.at[pl.ds(o, BLK)], prec_v,
                                      sem)

                def uloop(g, uc):
                    gb = g * LANES
                    sd = src_v[pl.ds(gb, LANES)]
                    lane = bi * BLK + gb + iota
                    m = lane < tvec
                    sd = jnp.where(m, sd,
                                   RNG + (g % 64) * 16 + iota)
                    src_v[pl.ds(gb, LANES)] = lax.shift_right_logical(sd, RSH)
                    dstl_v[pl.ds(gb, LANES)] = sd & (RPAD - 1)
                    return uc
                lax.fori_loop(0, NGRP, uloop, 0)

                d2 = pltpu.async_copy(h_hbm.at[src_v], xrows_v, sem)
                d1.wait(); d2.wait()

                def gloop(g, gc):
                    gb = g * LANES
                    for j in range(LANES):
                        e = gb + j
                        prow = prec_v[e, :]
                        pxs = jnp.take_along_axis(prow, z0, axis=0,
                                                  mode="promise_in_bounds")
                        pxd = jnp.take_along_axis(prow, z2, axis=0,
                                                  mode="promise_in_bounds")
                        pys = jnp.take_along_axis(prow, o1, axis=0,
                                                  mode="promise_in_bounds")
                        pyd = jnp.take_along_axis(prow, o3, axis=0,
                                                  mode="promise_in_bounds")
                        us = pxs - pxd
                        vs = pys - pyd
                        xr = xrows_v[e, :]
                        for k in range(4):
                            t = us * wa[k] + vs * wb[k] + bb[k]
                            t = jnp.maximum(t, 0.0)
                            msg_v[e, pl.ds(16 * k, 16)] = t * xr
                    return gc
                lax.fori_loop(0, NGRP, gloop, 0)
                pltpu.sync_copy(msg_v, agg_sh.at[dstl_v], add=True)
                return carry
            lax.fori_loop(0, trips, bloop, 0)
        plsc.subcore_barrier()

        pltpu.sync_copy(agg_sh.at[pl.ds(s * ROWS_PT, ROWS_PT)],
                        out_hbm.at[r, pl.ds(s * ROWS_PT, ROWS_PT)])
        plsc.subcore_barrier()


def _sc_layer(h, edge_lists, w_flat, b_in):
    sdl, prl, tot = edge_lists
    mesh = plsc.VectorSubcoreMesh(core_axis_name="c", subcore_axis_name="s",
                                  num_cores=NCORES, num_subcores=NSUB)
    kfn = pl.kernel(
        _sc_layer_body,
        out_type=jax.ShapeDtypeStruct((NRANGE, RPAD, H), jnp.float32),
        mesh=mesh,
        compiler_params=pltpu.CompilerParams(use_tc_tiling_on_sc=False),
        scratch_types=[
            pltpu.VMEM_SHARED((RPAD, H), jnp.float32),
            pltpu.VMEM((BLK,), jnp.int32),
            pltpu.VMEM((BLK,), jnp.int32),
            pltpu.VMEM((BLK, PW), jnp.float32),
            pltpu.VMEM((BLK, D), jnp.float32),
            pltpu.VMEM((BLK, H), jnp.float32),
            pltpu.VMEM((2 * H,), jnp.float32),
            pltpu.VMEM((H,), jnp.float32),
            pltpu.VMEM((LANES,), jnp.int32),
            pltpu.SemaphoreType.DMA,
        ],
    )
    return kfn(h, sdl, prl, tot, w_flat, b_in)


def _tc_affine(agg2, w_out, b_out2):
    AB = 512

    def body(agg_ref, w_ref, b_ref, out_ref):
        out_ref[...] = (
            jnp.dot(agg_ref[...], w_ref[...],
                    preferred_element_type=jnp.float32,
                    precision=lax.Precision.HIGHEST)
            + b_ref[...])

    return pl.pallas_call(
        body,
        grid=(NPAD // AB,),
        in_specs=[
            pl.BlockSpec((AB, H), lambda i: (i, 0)),
            pl.BlockSpec((H, D), lambda i: (0, 0)),
            pl.BlockSpec((1, D), lambda i: (0, 0)),
        ],
        out_specs=pl.BlockSpec((AB, D), lambda i: (i, 0)),
        out_shape=jax.ShapeDtypeStruct((NPAD, D), jnp.float32),
    )(agg2, w_out, b_out2)


def _tc_head(h, batch3, w_lin, b_lin2):
    BN = 1000
    NB = N // BN

    def body(h_ref, b_ref, wl_ref, bl_ref, out_ref, acc, cnt):
        i = pl.program_id(0)

        @pl.when(i == 0)
        def _init():
            acc[...] = jnp.zeros_like(acc)
            cnt[...] = jnp.zeros_like(cnt)

        gi = lax.broadcasted_iota(jnp.int32, (G, BN), 0)
        oht = (b_ref[0] == gi).astype(jnp.float32)
        acc[...] += lax.dot_general(oht, h_ref[...],
                                    (((1,), (0,)), ((), ())),
                                    preferred_element_type=jnp.float32,
                                    precision=lax.Precision.HIGHEST)
        cnt[...] += jnp.sum(oht, axis=1, keepdims=True)

        @pl.when(i == NB - 1)
        def _fin():
            pooled = acc[...] / jnp.maximum(cnt[...], 1.0)
            logits = (jnp.dot(pooled, wl_ref[...],
                              preferred_element_type=jnp.float32,
                              precision=lax.Precision.HIGHEST)
                      + bl_ref[...])
            mx = jnp.max(logits, axis=0, keepdims=True)
            z = logits - mx
            lse = jnp.log(jnp.sum(jnp.exp(z), axis=0, keepdims=True))
            out_ref[...] = z - lse

    return pl.pallas_call(
        body,
        grid=(NB,),
        in_specs=[
            pl.BlockSpec((BN, D), lambda i: (i, 0)),
            pl.BlockSpec((1, 1, BN), lambda i: (i, 0, 0)),
            pl.BlockSpec((D, NCLS), lambda i: (0, 0)),
            pl.BlockSpec((1, NCLS), lambda i: (0, 0)),
        ],
        out_specs=pl.BlockSpec((G, NCLS), lambda i: (0, 0)),
        out_shape=jax.ShapeDtypeStruct((G, NCLS), jnp.float32),
        scratch_shapes=[pltpu.VMEM((G, D), jnp.float32),
                        pltpu.VMEM((G, 1), jnp.float32)],
    )(h, batch3, w_lin, b_lin2)


def kernel(x, pos, edge_index, batch,
           W_in1, b_in1, W_out1, b_out1,
           W_in2, b_in2, W_out2, b_out2,
           W_in3, b_in3, W_out3, b_out3,
           W_lin, b_lin):
    srcp = edge_index[0] + (edge_index[0] // RNG) * (RPAD - RNG)
    dstp = edge_index[1] + (edge_index[1] // RNG) * (RPAD - RNG)
    pos_pad = jnp.pad(pos.reshape(NRANGE, RNG, 2),
                      ((0, 0), (0, RPAD - RNG), (0, PW - 2)))
    pos_xy8 = pos_pad.reshape(NPAD, PW)
    x_pad = jnp.pad(x.reshape(NRANGE, RNG, D),
                    ((0, 0), (0, RPAD - RNG), (0, 0))).reshape(NPAD, D)
    batch3 = batch.reshape(N // 1000, 1, 1000)

    edge_lists = _bin_edges(pos_xy8, srcp, dstp)

    hp = x_pad
    for (wi, bi, wo, bo) in ((W_in1, b_in1, W_out1, b_out1),
                             (W_in2, b_in2, W_out2, b_out2),
                             (W_in3, b_in3, W_out3, b_out3)):
        agg = _sc_layer(hp, edge_lists, wi.reshape(2 * H), bi)
        hp = _tc_affine(agg.reshape(NPAD, H), wo, bo.reshape(1, D))
    h = hp.reshape(NRANGE, RPAD, D)[:, :RNG, :].reshape(N, D)
    return _tc_head(h, batch3, W_lin, b_lin.reshape(1, NCLS))


# speedup vs baseline: 7.2745x; 1.0041x over previous
import jax
import jax.numpy as jnp
from jax import lax
from jax.experimental import pallas as pl
from jax.experimental.pallas import tpu as pltpu
from jax.experimental.pallas import tpu_sc as plsc

N = 100000
E = 3200000
D = 16
H = 64
G = 64
NCLS = 10

NCORES = 2
NSUB = 16
LANES = 16

NRANGE = 8
NPASS = NRANGE // NCORES
RNG = N // NRANGE
RPAD = 16384
RSH = 14
NPAD = NRANGE * RPAD
EPT = E // NSUB
EPC = EPT // NCORES

BLK = 512
NGRP = BLK // LANES
CAP = (EPC // BLK + 1) * BLK + BLK
ROWS_PT = RPAD // NSUB

BI = 800
NBI = EPC // BI
NGI = BI // LANES
PW = 16


def _lane_prefix_incl(v, iota):
    for dlt in (1, 2, 4, 8):
        idx = jnp.maximum(iota - dlt, 0)
        sh = jnp.take_along_axis(v, idx, axis=0, mode="promise_in_bounds")
        v = v + jnp.where(iota >= dlt, sh, 0)
    return v


def _bin_body(pxy_hbm, src_hbm, dst_hbm,
              sdl_hbm, prl_hbm, tot_hbm,
              srcb0, srcb1, dstb0, dstb1, slot_v, sd_v,
              psr0, psr1, pdr0, pdr1, prec, tv,
              semg0, semg1, sems):
    c = lax.axis_index("c")
    s = lax.axis_index("s")
    iota = lax.iota(jnp.int32, LANES)
    eoff0 = s * EPT + c * EPC
    dstb = dstb0

    def abloop(bi, carry):
        pltpu.sync_copy(dst_hbm.at[pl.ds(eoff0 + bi * BI, BI)], dstb)

        def agloop(g, cnts):
            d16 = dstb[pl.ds(g * LANES, LANES)]
            r16 = lax.shift_right_logical(d16, RSH)
            return tuple(cnts[r] + jnp.where(r16 == r, 1, 0)
                         for r in range(NRANGE))
        return lax.fori_loop(0, NGI, agloop, carry)

    zero16 = jnp.zeros((LANES,), jnp.int32)
    cnts = lax.fori_loop(0, NBI, abloop, (zero16,) * NRANGE)

    base_vecs = []
    for r in range(NRANGE):
        flatbase = ((c * NSUB + s) * NRANGE + r) * CAP
        incl = _lane_prefix_incl(cnts[r], iota)
        base_vecs.append(flatbase + incl - cnts[r])
        tot = jnp.take_along_axis(incl, jnp.full((LANES,), 15, jnp.int32),
                                  axis=0, mode="promise_in_bounds")
        tv[...] = tot
        pltpu.sync_copy(tv, tot_hbm.at[c, s, r])

    rot2 = (iota - 2) & (LANES - 1)
    msel = iota < 2

    bufs = ((srcb0, dstb0, psr0, pdr0, semg0),
            (srcb1, dstb1, psr1, pdr1, semg1))

    def issue(bi, bset):
        sb, db, ps, pd, sg = bset
        pltpu.sync_copy(src_hbm.at[pl.ds(eoff0 + bi * BI, BI)], sb)
        pltpu.sync_copy(dst_hbm.at[pl.ds(eoff0 + bi * BI, BI)], db)
        pltpu.async_copy(pxy_hbm.at[sb], ps, sg)
        pltpu.async_copy(pxy_hbm.at[db], pd, sg)

    def step(bi, bset, nxt_bi, nxt_bset, carry, prefetch):
        sb, db, ps, pd, sg = bset
        pltpu.make_async_copy(pxy_hbm.at[sb], ps, sg).wait()
        pltpu.make_async_copy(pxy_hbm.at[db], pd, sg).wait()
        if prefetch:
            issue(nxt_bi, nxt_bset)

        def bgloop(g, cnts):
            gb = g * LANES
            s16 = sb[pl.ds(gb, LANES)]
            d16 = db[pl.ds(gb, LANES)]
            r16 = lax.shift_right_logical(d16, RSH)
            sd_v[pl.ds(gb, LANES)] = (
                lax.shift_left(s16, RSH) | (d16 & (RPAD - 1)))
            slot = zero16
            ncnts = []
            for r in range(NRANGE):
                mr = r16 == r
                slot = jnp.where(mr, base_vecs[r] + cnts[r], slot)
                ncnts.append(cnts[r] + jnp.where(mr, 1, 0))
            slot_v[pl.ds(gb, LANES)] = slot
            for j in range(LANES):
                e = gb + j
                sp = ps[e, :]
                dp = pd[e, :]
                dp2 = jnp.take_along_axis(dp, rot2, axis=0,
                                          mode="promise_in_bounds")
                prec[e, :] = jnp.where(msel, sp, dp2)
            return tuple(ncnts)
        carry = lax.fori_loop(0, NGI, bgloop, carry)
        e1 = pltpu.async_copy(sd_v, sdl_hbm.at[slot_v], sems)
        e2 = pltpu.async_copy(prec, prl_hbm.at[slot_v], sems)
        e1.wait(); e2.wait()
        return carry

    issue(0, bufs[0])

    def bbpair(bi2, carry):
        b0 = bi2 * 2
        carry = step(b0, bufs[0], b0 + 1, bufs[1], carry, True)
        carry = step(b0 + 1, bufs[1],
                     jnp.minimum(b0 + 2, NBI - 1), bufs[0], carry, True)
        return carry

    carry = lax.fori_loop(0, (NBI - 1) // 2, bbpair, (zero16,) * NRANGE)
    step(NBI - 1, bufs[0], 0, bufs[1], carry, False)


def _bin_edges(pos_xy8, srcp, dstp):
    mesh = plsc.VectorSubcoreMesh(core_axis_name="c", subcore_axis_name="s",
                                  num_cores=NCORES, num_subcores=NSUB)
    nlist = NCORES * NSUB * NRANGE * CAP
    kfn = pl.kernel(
        _bin_body,
        out_type=(
            jax.ShapeDtypeStruct((nlist,), jnp.int32),
            jax.ShapeDtypeStruct((nlist, PW), jnp.float32),
            jax.ShapeDtypeStruct((NCORES, NSUB, NRANGE, LANES), jnp.int32),
        ),
        mesh=mesh,
        compiler_params=pltpu.CompilerParams(use_tc_tiling_on_sc=False),
        scratch_types=[
            pltpu.VMEM((BI,), jnp.int32),
            pltpu.VMEM((BI,), jnp.int32),
            pltpu.VMEM((BI,), jnp.int32),
            pltpu.VMEM((BI,), jnp.int32),
            pltpu.VMEM((BI,), jnp.int32),
            pltpu.VMEM((BI,), jnp.int32),
            pltpu.VMEM((BI, PW), jnp.float32),
            pltpu.VMEM((BI, PW), jnp.float32),
            pltpu.VMEM((BI, PW), jnp.float32),
            pltpu.VMEM((BI, PW), jnp.float32),
            pltpu.VMEM((BI, PW), jnp.float32),
            pltpu.VMEM((LANES,), jnp.int32),
            pltpu.SemaphoreType.DMA,
            pltpu.SemaphoreType.DMA,
            pltpu.SemaphoreType.DMA,
        ],
    )
    return kfn(pos_xy8, srcp, dstp)


def _sc_layer_body(h_hbm, sdl_hbm, prl_hbm, tot_hbm,
                   w_hbm, b_hbm, out_hbm,
                   agg_sh, src_v, dstl_v, prec_v, xrows_v, msg_v,
                   w_v, b_v, tv, sem):
    c = lax.axis_index("c")
    s = lax.axis_index("s")
    iota = lax.iota(jnp.int32, LANES)
    pltpu.sync_copy(w_hbm, w_v)
    pltpu.sync_copy(b_hbm, b_v)
    wa = [w_v[pl.ds(16 * k, 16)] for k in range(4)]
    wb = [w_v[pl.ds(H + 16 * k, 16)] for k in range(4)]
    bb = [b_v[pl.ds(16 * k, 16)] for k in range(4)]
    z0 = jnp.full((LANES,), 0, jnp.int32)
    z2 = jnp.full((LANES,), 2, jnp.int32)
    o1 = jnp.full((LANES,), 1, jnp.int32)
    o3 = jnp.full((LANES,), 3, jnp.int32)

    for p in range(NPASS):
        r = NPASS * c + p

        def zloop(e, carry):
            for k in range(4):
                msg_v[e, pl.ds(16 * k, 16)] = jnp.zeros((16,), jnp.float32)
            return carry
        lax.fori_loop(0, BLK, zloop, 0)
        for q in range(ROWS_PT // BLK):
            pltpu.sync_copy(msg_v, agg_sh.at[pl.ds(s * ROWS_PT + q * BLK, BLK)])
        plsc.subcore_barrier()

        for cc in range(NCORES):
            flatbase = ((cc * NSUB + s) * NRANGE + r) * CAP
            pltpu.sync_copy(tot_hbm.at[cc, s, r], tv)
            tvec = tv[...]
            total = tvec[0]
            trips = lax.shift_right_logical(total + (BLK - 1), 9)

            def bloop(bi, carry, flatbase=flatbase, tvec=tvec, total=total):
                o = flatbase + bi * BLK
                pltpu.sync_copy(sdl_hbm.at[pl.ds(o, BLK)], src_v)
                d1 = pltpu.async_copy(prl_hbm.at[pl.ds(o, BLK)], prec_v,
                                      sem)

                def uloop(g, uc):
                    gb = g * LANES
                    sd = src_v[pl.ds(gb, LANES)]
                    lane = bi * BLK + gb + iota
                    m = lane < tvec
                    sd = jnp.where(m, sd,
                                   RNG + (g % 64) * 16 + iota)
                    src_v[pl.ds(gb, LANES)] = lax.shift_right_logical(sd, RSH)
                    dstl_v[pl.ds(gb, LANES)] = sd & (RPAD - 1)
                    return uc
                lax.fori_loop(0, NGRP, uloop, 0)

                d2 = pltpu.async_copy(h_hbm.at[src_v], xrows_v, sem)
                d1.wait(); d2.wait()

                def gloop(g, gc):
                    gb = g * LANES
                    for j in range(LANES):
                        e = gb + j
                        prow = prec_v[e, :]
                        pxs = jnp.take_along_axis(prow, z0, axis=0,
                                                  mode="promise_in_bounds")
                        pxd = jnp.take_along_axis(prow, z2, axis=0,
                                                  mode="promise_in_bounds")
                        pys = jnp.take_along_axis(prow, o1, axis=0,
                                                  mode="promise_in_bounds")
                        pyd = jnp.take_along_axis(prow, o3, axis=0,
                                                  mode="promise_in_bounds")
                        us = pxs - pxd
                        vs = pys - pyd
                        xr = xrows_v[e, :]
                        for k in range(4):
                            t = us * wa[k] + vs * wb[k] + bb[k]
                            t = jnp.maximum(t, 0.0)
                            msg_v[e, pl.ds(16 * k, 16)] = t * xr
                    return gc
                lax.fori_loop(0, NGRP, gloop, 0)
                pltpu.sync_copy(msg_v, agg_sh.at[dstl_v], add=True)
                return carry
            lax.fori_loop(0, trips, bloop, 0)
        plsc.subcore_barrier()

        pltpu.sync_copy(agg_sh.at[pl.ds(s * ROWS_PT, ROWS_PT)],
                        out_hbm.at[r, pl.ds(s * ROWS_PT, ROWS_PT)])
        plsc.subcore_barrier()


def _sc_layer(h, edge_lists, w_flat, b_in):
    sdl, prl, tot = edge_lists
    mesh = plsc.VectorSubcoreMesh(core_axis_name="c", subcore_axis_name="s",
                                  num_cores=NCORES, num_subcores=NSUB)
    kfn = pl.kernel(
        _sc_layer_body,
        out_type=jax.ShapeDtypeStruct((NRANGE, RPAD, H), jnp.float32),
        mesh=mesh,
        compiler_params=pltpu.CompilerParams(use_tc_tiling_on_sc=False),
        scratch_types=[
            pltpu.VMEM_SHARED((RPAD, H), jnp.float32),
            pltpu.VMEM((BLK,), jnp.int32),
            pltpu.VMEM((BLK,), jnp.int32),
            pltpu.VMEM((BLK, PW), jnp.float32),
            pltpu.VMEM((BLK, D), jnp.float32),
            pltpu.VMEM((BLK, H), jnp.float32),
            pltpu.VMEM((2 * H,), jnp.float32),
            pltpu.VMEM((H,), jnp.float32),
            pltpu.VMEM((LANES,), jnp.int32),
            pltpu.SemaphoreType.DMA,
        ],
    )
    return kfn(h, sdl, prl, tot, w_flat, b_in)


def _tc_affine(agg2, w_out, b_out2):
    AB = 512

    def body(agg_ref, w_ref, b_ref, out_ref):
        out_ref[...] = (
            jnp.dot(agg_ref[...], w_ref[...],
                    preferred_element_type=jnp.float32,
                    precision=lax.Precision.HIGHEST)
            + b_ref[...])

    return pl.pallas_call(
        body,
        grid=(NPAD // AB,),
        in_specs=[
            pl.BlockSpec((AB, H), lambda i: (i, 0)),
            pl.BlockSpec((H, D), lambda i: (0, 0)),
            pl.BlockSpec((1, D), lambda i: (0, 0)),
        ],
        out_specs=pl.BlockSpec((AB, D), lambda i: (i, 0)),
        out_shape=jax.ShapeDtypeStruct((NPAD, D), jnp.float32),
    )(agg2, w_out, b_out2)


def _tc_head(h, batch3, w_lin, b_lin2):
    BN = 1000
    NB = N // BN

    def body(h_ref, b_ref, wl_ref, bl_ref, out_ref, acc, cnt):
        i = pl.program_id(0)

        @pl.when(i == 0)
        def _init():
            acc[...] = jnp.zeros_like(acc)
            cnt[...] = jnp.zeros_like(cnt)

        gi = lax.broadcasted_iota(jnp.int32, (G, BN), 0)
        oht = (b_ref[0] == gi).astype(jnp.float32)
        acc[...] += lax.dot_general(oht, h_ref[...],
                                    (((1,), (0,)), ((), ())),
                                    preferred_element_type=jnp.float32,
                                    precision=lax.Precision.HIGHEST)
        cnt[...] += jnp.sum(oht, axis=1, keepdims=True)

        @pl.when(i == NB - 1)
        def _fin():
            pooled = acc[...] / jnp.maximum(cnt[...], 1.0)
            logits = (jnp.dot(pooled, wl_ref[...],
                              preferred_element_type=jnp.float32,
                              precision=lax.Precision.HIGHEST)
                      + bl_ref[...])
            mx = jnp.max(logits, axis=0, keepdims=True)
            z = logits - mx
            lse = jnp.log(jnp.sum(jnp.exp(z), axis=0, keepdims=True))
            out_ref[...] = z - lse

    return pl.pallas_call(
        body,
        grid=(NB,),
        in_specs=[
            pl.BlockSpec((BN, D), lambda i: (i, 0)),
            pl.BlockSpec((1, 1, BN), lambda i: (i, 0, 0)),
            pl.BlockSpec((D, NCLS), lambda i: (0, 0)),
            pl.BlockSpec((1, NCLS), lambda i: (0, 0)),
        ],
        out_specs=pl.BlockSpec((G, NCLS), lambda i: (0, 0)),
        out_shape=jax.ShapeDtypeStruct((G, NCLS), jnp.float32),
        scratch_shapes=[pltpu.VMEM((G, D), jnp.float32),
                        pltpu.VMEM((G, 1), jnp.float32)],
    )(h, batch3, w_lin, b_lin2)


def kernel(x, pos, edge_index, batch,
           W_in1, b_in1, W_out1, b_out1,
           W_in2, b_in2, W_out2, b_out2,
           W_in3, b_in3, W_out3, b_out3,
           W_lin, b_lin):
    srcp = edge_index[0] + (edge_index[0] // RNG) * (RPAD - RNG)
    dstp = edge_index[1] + (edge_index[1] // RNG) * (RPAD - RNG)
    pos_pad = jnp.pad(pos.reshape(NRANGE, RNG, 2),
                      ((0, 0), (0, RPAD - RNG), (0, PW - 2)))
    pos_xy8 = pos_pad.reshape(NPAD, PW)
    x_pad = jnp.pad(x.reshape(NRANGE, RNG, D),
                    ((0, 0), (0, RPAD - RNG), (0, 0))).reshape(NPAD, D)
    batch3 = batch.reshape(N // 1000, 1, 1000)

    edge_lists = _bin_edges(pos_xy8, srcp, dstp)

    hp = x_pad
    for (wi, bi, wo, bo) in ((W_in1, b_in1, W_out1, b_out1),
                             (W_in2, b_in2, W_out2, b_out2),
                             (W_in3, b_in3, W_out3, b_out3)):
        agg = _sc_layer(hp, edge_lists, wi.reshape(2 * H), bi)
        hp = _tc_affine(agg.reshape(NPAD, H), wo, bo.reshape(1, D))
    h = hp.reshape(NRANGE, RPAD, D)[:, :RNG, :].reshape(N, D)
    return _tc_head(h, batch3, W_lin, b_lin.reshape(1, NCLS))


# speedup vs baseline: 7.5845x; 1.0426x over previous
import jax
import jax.numpy as jnp
from jax import lax
from jax.experimental import pallas as pl
from jax.experimental.pallas import tpu as pltpu
from jax.experimental.pallas import tpu_sc as plsc

N = 100000
E = 3200000
D = 16
H = 64
G = 64
NCLS = 10

NCORES = 2
NSUB = 16
LANES = 16

NRANGE = 8
NPASS = NRANGE // NCORES
RNG = N // NRANGE
RPAD = 16384
RSH = 14
NPAD = NRANGE * RPAD
EPT = E // NSUB
EPC = EPT // NCORES

BLK = 512
NGRP = BLK // LANES
CAP = (EPC // BLK + 1) * BLK + BLK
ROWS_PT = RPAD // NSUB

BI = 800
NBI = EPC // BI
NGI = BI // LANES
PW = 16


def _lane_prefix_incl(v, iota):
    for dlt in (1, 2, 4, 8):
        idx = jnp.maximum(iota - dlt, 0)
        sh = jnp.take_along_axis(v, idx, axis=0, mode="promise_in_bounds")
        v = v + jnp.where(iota >= dlt, sh, 0)
    return v


def _bin_body(pxy_hbm, src_hbm, dst_hbm,
              sdl_hbm, prl_hbm, tot_hbm,
              srcb0, srcb1, dstb0, dstb1, slot_v, sd_v,
              psr0, psr1, pdr0, pdr1, prec, tv,
              semg0, semg1, sems):
    c = lax.axis_index("c")
    s = lax.axis_index("s")
    iota = lax.iota(jnp.int32, LANES)
    eoff0 = s * EPT + c * EPC
    dstb = dstb0

    def abloop(bi, carry):
        pltpu.sync_copy(dst_hbm.at[pl.ds(eoff0 + bi * BI, BI)], dstb)

        def agloop(g, cnts):
            d16 = dstb[pl.ds(g * LANES, LANES)]
            r16 = lax.shift_right_logical(d16, RSH)
            return tuple(cnts[r] + jnp.where(r16 == r, 1, 0)
                         for r in range(NRANGE))
        return lax.fori_loop(0, NGI, agloop, carry)

    zero16 = jnp.zeros((LANES,), jnp.int32)
    cnts = lax.fori_loop(0, NBI, abloop, (zero16,) * NRANGE)

    base_vecs = []
    for r in range(NRANGE):
        flatbase = ((c * NSUB + s) * NRANGE + r) * CAP
        incl = _lane_prefix_incl(cnts[r], iota)
        base_vecs.append(flatbase + incl - cnts[r])
        tot = jnp.take_along_axis(incl, jnp.full((LANES,), 15, jnp.int32),
                                  axis=0, mode="promise_in_bounds")
        tv[...] = tot
        pltpu.sync_copy(tv, tot_hbm.at[c, s, r])


    bufs = ((srcb0, dstb0, psr0, pdr0, semg0),
            (srcb1, dstb1, psr1, pdr1, semg1))

    def issue(bi, bset):
        sb, db, ps, pd, sg = bset
        pltpu.sync_copy(src_hbm.at[pl.ds(eoff0 + bi * BI, BI)], sb)
        pltpu.sync_copy(dst_hbm.at[pl.ds(eoff0 + bi * BI, BI)], db)
        pltpu.async_copy(pxy_hbm.at[sb], ps, sg)
        pltpu.async_copy(pxy_hbm.at[db], pd, sg)

    def step(bi, bset, nxt_bi, nxt_bset, carry, prefetch):
        sb, db, ps, pd, sg = bset
        pltpu.make_async_copy(pxy_hbm.at[sb], ps, sg).wait()
        pltpu.make_async_copy(pxy_hbm.at[db], pd, sg).wait()
        if prefetch:
            issue(nxt_bi, nxt_bset)

        def bgloop(g, cnts):
            gb = g * LANES
            s16 = sb[pl.ds(gb, LANES)]
            d16 = db[pl.ds(gb, LANES)]
            r16 = lax.shift_right_logical(d16, RSH)
            sd_v[pl.ds(gb, LANES)] = (
                lax.shift_left(s16, RSH) | (d16 & (RPAD - 1)))
            slot = zero16
            ncnts = []
            for r in range(NRANGE):
                mr = r16 == r
                slot = jnp.where(mr, base_vecs[r] + cnts[r], slot)
                ncnts.append(cnts[r] + jnp.where(mr, 1, 0))
            slot_v[pl.ds(gb, LANES)] = slot
            for j in range(LANES):
                e = gb + j
                prec[e, :] = ps[e, :] - pd[e, :]
            return tuple(ncnts)
        carry = lax.fori_loop(0, NGI, bgloop, carry)
        e1 = pltpu.async_copy(sd_v, sdl_hbm.at[slot_v], sems)
        e2 = pltpu.async_copy(prec, prl_hbm.at[slot_v], sems)
        e1.wait(); e2.wait()
        return carry

    issue(0, bufs[0])

    def bbpair(bi2, carry):
        b0 = bi2 * 2
        carry = step(b0, bufs[0], b0 + 1, bufs[1], carry, True)
        carry = step(b0 + 1, bufs[1],
                     jnp.minimum(b0 + 2, NBI - 1), bufs[0], carry, True)
        return carry

    carry = lax.fori_loop(0, (NBI - 1) // 2, bbpair, (zero16,) * NRANGE)
    step(NBI - 1, bufs[0], 0, bufs[1], carry, False)


def _bin_edges(pos_xy8, srcp, dstp):
    mesh = plsc.VectorSubcoreMesh(core_axis_name="c", subcore_axis_name="s",
                                  num_cores=NCORES, num_subcores=NSUB)
    nlist = NCORES * NSUB * NRANGE * CAP
    kfn = pl.kernel(
        _bin_body,
        out_type=(
            jax.ShapeDtypeStruct((nlist,), jnp.int32),
            jax.ShapeDtypeStruct((nlist, PW), jnp.float32),
            jax.ShapeDtypeStruct((NCORES, NSUB, NRANGE, LANES), jnp.int32),
        ),
        mesh=mesh,
        compiler_params=pltpu.CompilerParams(use_tc_tiling_on_sc=False),
        scratch_types=[
            pltpu.VMEM((BI,), jnp.int32),
            pltpu.VMEM((BI,), jnp.int32),
            pltpu.VMEM((BI,), jnp.int32),
            pltpu.VMEM((BI,), jnp.int32),
            pltpu.VMEM((BI,), jnp.int32),
            pltpu.VMEM((BI,), jnp.int32),
            pltpu.VMEM((BI, PW), jnp.float32),
            pltpu.VMEM((BI, PW), jnp.float32),
            pltpu.VMEM((BI, PW), jnp.float32),
            pltpu.VMEM((BI, PW), jnp.float32),
            pltpu.VMEM((BI, PW), jnp.float32),
            pltpu.VMEM((LANES,), jnp.int32),
            pltpu.SemaphoreType.DMA,
            pltpu.SemaphoreType.DMA,
            pltpu.SemaphoreType.DMA,
        ],
    )
    return kfn(pos_xy8, srcp, dstp)


def _sc_layer_body(h_hbm, sdl_hbm, prl_hbm, tot_hbm,
                   w_hbm, b_hbm, out_hbm,
                   agg_sh, src_v, dstl_v, prec_v, xrows_v, msg_v,
                   w_v, b_v, tv, sem):
    c = lax.axis_index("c")
    s = lax.axis_index("s")
    iota = lax.iota(jnp.int32, LANES)
    pltpu.sync_copy(w_hbm, w_v)
    pltpu.sync_copy(b_hbm, b_v)
    wa = [w_v[pl.ds(16 * k, 16)] for k in range(4)]
    wb = [w_v[pl.ds(H + 16 * k, 16)] for k in range(4)]
    bb = [b_v[pl.ds(16 * k, 16)] for k in range(4)]
    z0 = jnp.full((LANES,), 0, jnp.int32)
    o1 = jnp.full((LANES,), 1, jnp.int32)

    for p in range(NPASS):
        r = NPASS * c + p

        def zloop(e, carry):
            for k in range(4):
                msg_v[e, pl.ds(16 * k, 16)] = jnp.zeros((16,), jnp.float32)
            return carry
        lax.fori_loop(0, BLK, zloop, 0)
        for q in range(ROWS_PT // BLK):
            pltpu.sync_copy(msg_v, agg_sh.at[pl.ds(s * ROWS_PT + q * BLK, BLK)])
        plsc.subcore_barrier()

        for cc in range(NCORES):
            flatbase = ((cc * NSUB + s) * NRANGE + r) * CAP
            pltpu.sync_copy(tot_hbm.at[cc, s, r], tv)
            tvec = tv[...]
            total = tvec[0]
            trips = lax.shift_right_logical(total + (BLK - 1), 9)

            def bloop(bi, carry, flatbase=flatbase, tvec=tvec, total=total):
                o = flatbase + bi * BLK
                pltpu.sync_copy(sdl_hbm.at[pl.ds(o, BLK)], src_v)
                d1 = pltpu.async_copy(prl_hbm.at[pl.ds(o, BLK)], prec_v,
                                      sem)

                def uloop(g, uc):
                    gb = g * LANES
                    sd = src_v[pl.ds(gb, LANES)]
                    lane = bi * BLK + gb + iota
                    m = lane < tvec
                    sd = jnp.where(m, sd,
                                   RNG + (g % 64) * 16 + iota)
                    src_v[pl.ds(gb, LANES)] = lax.shift_right_logical(sd, RSH)
                    dstl_v[pl.ds(gb, LANES)] = sd & (RPAD - 1)
                    return uc
                lax.fori_loop(0, NGRP, uloop, 0)

                d2 = pltpu.async_copy(h_hbm.at[src_v], xrows_v, sem)
                d1.wait(); d2.wait()

                def gloop(g, gc):
                    gb = g * LANES
                    for j in range(LANES):
                        e = gb + j
                        prow = prec_v[e, :]
                        us = jnp.take_along_axis(prow, z0, axis=0,
                                                 mode="promise_in_bounds")
                        vs = jnp.take_along_axis(prow, o1, axis=0,
                                                 mode="promise_in_bounds")
                        xr = xrows_v[e, :]
                        for k in range(4):
                            t = us * wa[k] + vs * wb[k] + bb[k]
                            t = jnp.maximum(t, 0.0)
                            msg_v[e, pl.ds(16 * k, 16)] = t * xr
                    return gc
                lax.fori_loop(0, NGRP, gloop, 0)
                pltpu.sync_copy(msg_v, agg_sh.at[dstl_v], add=True)
                return carry
            lax.fori_loop(0, trips, bloop, 0)
        plsc.subcore_barrier()

        pltpu.sync_copy(agg_sh.at[pl.ds(s * ROWS_PT, ROWS_PT)],
                        out_hbm.at[r, pl.ds(s * ROWS_PT, ROWS_PT)])
        plsc.subcore_barrier()


def _sc_layer(h, edge_lists, w_flat, b_in):
    sdl, prl, tot = edge_lists
    mesh = plsc.VectorSubcoreMesh(core_axis_name="c", subcore_axis_name="s",
                                  num_cores=NCORES, num_subcores=NSUB)
    kfn = pl.kernel(
        _sc_layer_body,
        out_type=jax.ShapeDtypeStruct((NRANGE, RPAD, H), jnp.float32),
        mesh=mesh,
        compiler_params=pltpu.CompilerParams(use_tc_tiling_on_sc=False),
        scratch_types=[
            pltpu.VMEM_SHARED((RPAD, H), jnp.float32),
            pltpu.VMEM((BLK,), jnp.int32),
            pltpu.VMEM((BLK,), jnp.int32),
            pltpu.VMEM((BLK, PW), jnp.float32),
            pltpu.VMEM((BLK, D), jnp.float32),
            pltpu.VMEM((BLK, H), jnp.float32),
            pltpu.VMEM((2 * H,), jnp.float32),
            pltpu.VMEM((H,), jnp.float32),
            pltpu.VMEM((LANES,), jnp.int32),
            pltpu.SemaphoreType.DMA,
        ],
    )
    return kfn(h, sdl, prl, tot, w_flat, b_in)


def _tc_affine(agg2, w_out, b_out2):
    AB = 512

    def body(agg_ref, w_ref, b_ref, out_ref):
        out_ref[...] = (
            jnp.dot(agg_ref[...], w_ref[...],
                    preferred_element_type=jnp.float32,
                    precision=lax.Precision.HIGHEST)
            + b_ref[...])

    return pl.pallas_call(
        body,
        grid=(NPAD // AB,),
        in_specs=[
            pl.BlockSpec((AB, H), lambda i: (i, 0)),
            pl.BlockSpec((H, D), lambda i: (0, 0)),
            pl.BlockSpec((1, D), lambda i: (0, 0)),
        ],
        out_specs=pl.BlockSpec((AB, D), lambda i: (i, 0)),
        out_shape=jax.ShapeDtypeStruct((NPAD, D), jnp.float32),
    )(agg2, w_out, b_out2)


def _tc_head(h, batch3, w_lin, b_lin2):
    BN = 1000
    NB = N // BN

    def body(h_ref, b_ref, wl_ref, bl_ref, out_ref, acc, cnt):
        i = pl.program_id(0)

        @pl.when(i == 0)
        def _init():
            acc[...] = jnp.zeros_like(acc)
            cnt[...] = jnp.zeros_like(cnt)

        gi = lax.broadcasted_iota(jnp.int32, (G, BN), 0)
        oht = (b_ref[0] == gi).astype(jnp.float32)
        acc[...] += lax.dot_general(oht, h_ref[...],
                                    (((1,), (0,)), ((), ())),
                                    preferred_element_type=jnp.float32,
                                    precision=lax.Precision.HIGHEST)
        cnt[...] += jnp.sum(oht, axis=1, keepdims=True)

        @pl.when(i == NB - 1)
        def _fin():
            pooled = acc[...] / jnp.maximum(cnt[...], 1.0)
            logits = (jnp.dot(pooled, wl_ref[...],
                              preferred_element_type=jnp.float32,
                              precision=lax.Precision.HIGHEST)
                      + bl_ref[...])
            mx = jnp.max(logits, axis=0, keepdims=True)
            z = logits - mx
            lse = jnp.log(jnp.sum(jnp.exp(z), axis=0, keepdims=True))
            out_ref[...] = z - lse

    return pl.pallas_call(
        body,
        grid=(NB,),
        in_specs=[
            pl.BlockSpec((BN, D), lambda i: (i, 0)),
            pl.BlockSpec((1, 1, BN), lambda i: (i, 0, 0)),
            pl.BlockSpec((D, NCLS), lambda i: (0, 0)),
            pl.BlockSpec((1, NCLS), lambda i: (0, 0)),
        ],
        out_specs=pl.BlockSpec((G, NCLS), lambda i: (0, 0)),
        out_shape=jax.ShapeDtypeStruct((G, NCLS), jnp.float32),
        scratch_shapes=[pltpu.VMEM((G, D), jnp.float32),
                        pltpu.VMEM((G, 1), jnp.float32)],
    )(h, batch3, w_lin, b_lin2)


def kernel(x, pos, edge_index, batch,
           W_in1, b_in1, W_out1, b_out1,
           W_in2, b_in2, W_out2, b_out2,
           W_in3, b_in3, W_out3, b_out3,
           W_lin, b_lin):
    srcp = edge_index[0] + (edge_index[0] // RNG) * (RPAD - RNG)
    dstp = edge_index[1] + (edge_index[1] // RNG) * (RPAD - RNG)
    pos_pad = jnp.pad(pos.reshape(NRANGE, RNG, 2),
                      ((0, 0), (0, RPAD - RNG), (0, PW - 2)))
    pos_xy8 = pos_pad.reshape(NPAD, PW)
    x_pad = jnp.pad(x.reshape(NRANGE, RNG, D),
                    ((0, 0), (0, RPAD - RNG), (0, 0))).reshape(NPAD, D)
    batch3 = batch.reshape(N // 1000, 1, 1000)

    edge_lists = _bin_edges(pos_xy8, srcp, dstp)

    hp = x_pad
    for (wi, bi, wo, bo) in ((W_in1, b_in1, W_out1, b_out1),
                             (W_in2, b_in2, W_out2, b_out2),
                             (W_in3, b_in3, W_out3, b_out3)):
        agg = _sc_layer(hp, edge_lists, wi.reshape(2 * H), bi)
        hp = _tc_affine(agg.reshape(NPAD, H), wo, bo.reshape(1, D))
    h = hp.reshape(NRANGE, RPAD, D)[:, :RNG, :].reshape(N, D)
    return _tc_head(h, batch3, W_lin, b_lin.reshape(1, NCLS))


# speedup vs baseline: 8.4448x; 1.1134x over previous
import jax
import jax.numpy as jnp
from jax import lax
from jax.experimental import pallas as pl
from jax.experimental.pallas import tpu as pltpu
from jax.experimental.pallas import tpu_sc as plsc

N = 100000
E = 3200000
D = 16
H = 64
G = 64
NCLS = 10

NCORES = 2
NSUB = 16
LANES = 16

NRANGE = 8
NPASS = NRANGE // NCORES
RNG = N // NRANGE
RPAD = 16384
RSH = 14
NPAD = NRANGE * RPAD
EPT = E // NSUB
EPC = EPT // NCORES

BLK = 256
BSH = 8
NGRP = BLK // LANES
CAP = (EPC // BLK + 1) * BLK + BLK
ROWS_PT = RPAD // NSUB

BI = 800
NBI = EPC // BI
NGI = BI // LANES
PW = 16


def _lane_prefix_incl(v, iota):
    for dlt in (1, 2, 4, 8):
        idx = jnp.maximum(iota - dlt, 0)
        sh = jnp.take_along_axis(v, idx, axis=0, mode="promise_in_bounds")
        v = v + jnp.where(iota >= dlt, sh, 0)
    return v


def _bin_body(pxy_hbm, src_hbm, dst_hbm,
              sdl_hbm, prl_hbm, tot_hbm,
              srcb0, srcb1, dstb0, dstb1, slot_v, sd_v,
              psr0, psr1, pdr0, pdr1, prec, tv,
              semg0, semg1, sems):
    c = lax.axis_index("c")
    s = lax.axis_index("s")
    iota = lax.iota(jnp.int32, LANES)
    eoff0 = s * EPT + c * EPC
    dstb = dstb0

    def abloop(bi, carry):
        pltpu.sync_copy(dst_hbm.at[pl.ds(eoff0 + bi * BI, BI)], dstb)

        def agloop(g, cnts):
            d16 = dstb[pl.ds(g * LANES, LANES)]
            r16 = lax.shift_right_logical(d16, RSH)
            return tuple(cnts[r] + jnp.where(r16 == r, 1, 0)
                         for r in range(NRANGE))
        return lax.fori_loop(0, NGI, agloop, carry)

    zero16 = jnp.zeros((LANES,), jnp.int32)
    cnts = lax.fori_loop(0, NBI, abloop, (zero16,) * NRANGE)

    base_vecs = []
    for r in range(NRANGE):
        flatbase = ((c * NSUB + s) * NRANGE + r) * CAP
        incl = _lane_prefix_incl(cnts[r], iota)
        base_vecs.append(flatbase + incl - cnts[r])
        tot = jnp.take_along_axis(incl, jnp.full((LANES,), 15, jnp.int32),
                                  axis=0, mode="promise_in_bounds")
        tv[...] = tot
        pltpu.sync_copy(tv, tot_hbm.at[c, s, r])


    bufs = ((srcb0, dstb0, psr0, pdr0, semg0),
            (srcb1, dstb1, psr1, pdr1, semg1))

    def issue(bi, bset):
        sb, db, ps, pd, sg = bset
        pltpu.sync_copy(src_hbm.at[pl.ds(eoff0 + bi * BI, BI)], sb)
        pltpu.sync_copy(dst_hbm.at[pl.ds(eoff0 + bi * BI, BI)], db)
        pltpu.async_copy(pxy_hbm.at[sb], ps, sg)
        pltpu.async_copy(pxy_hbm.at[db], pd, sg)

    def step(bi, bset, nxt_bi, nxt_bset, carry, prefetch):
        sb, db, ps, pd, sg = bset
        pltpu.make_async_copy(pxy_hbm.at[sb], ps, sg).wait()
        pltpu.make_async_copy(pxy_hbm.at[db], pd, sg).wait()
        if prefetch:
            issue(nxt_bi, nxt_bset)

        def bgloop(g, cnts):
            gb = g * LANES
            s16 = sb[pl.ds(gb, LANES)]
            d16 = db[pl.ds(gb, LANES)]
            r16 = lax.shift_right_logical(d16, RSH)
            sd_v[pl.ds(gb, LANES)] = (
                lax.shift_left(s16, RSH) | (d16 & (RPAD - 1)))
            slot = zero16
            ncnts = []
            for r in range(NRANGE):
                mr = r16 == r
                slot = jnp.where(mr, base_vecs[r] + cnts[r], slot)
                ncnts.append(cnts[r] + jnp.where(mr, 1, 0))
            slot_v[pl.ds(gb, LANES)] = slot
            for j in range(LANES):
                e = gb + j
                prec[e, :] = ps[e, :] - pd[e, :]
            return tuple(ncnts)
        carry = lax.fori_loop(0, NGI, bgloop, carry)
        e1 = pltpu.async_copy(sd_v, sdl_hbm.at[slot_v], sems)
        e2 = pltpu.async_copy(prec, prl_hbm.at[slot_v], sems)
        e1.wait(); e2.wait()
        return carry

    issue(0, bufs[0])

    def bbpair(bi2, carry):
        b0 = bi2 * 2
        carry = step(b0, bufs[0], b0 + 1, bufs[1], carry, True)
        carry = step(b0 + 1, bufs[1],
                     jnp.minimum(b0 + 2, NBI - 1), bufs[0], carry, True)
        return carry

    carry = lax.fori_loop(0, (NBI - 1) // 2, bbpair, (zero16,) * NRANGE)
    step(NBI - 1, bufs[0], 0, bufs[1], carry, False)


def _bin_edges(pos_xy8, srcp, dstp):
    mesh = plsc.VectorSubcoreMesh(core_axis_name="c", subcore_axis_name="s",
                                  num_cores=NCORES, num_subcores=NSUB)
    nlist = NCORES * NSUB * NRANGE * CAP
    kfn = pl.kernel(
        _bin_body,
        out_type=(
            jax.ShapeDtypeStruct((nlist,), jnp.int32),
            jax.ShapeDtypeStruct((nlist, PW), jnp.float32),
            jax.ShapeDtypeStruct((NCORES, NSUB, NRANGE, LANES), jnp.int32),
        ),
        mesh=mesh,
        compiler_params=pltpu.CompilerParams(use_tc_tiling_on_sc=False),
        scratch_types=[
            pltpu.VMEM((BI,), jnp.int32),
            pltpu.VMEM((BI,), jnp.int32),
            pltpu.VMEM((BI,), jnp.int32),
            pltpu.VMEM((BI,), jnp.int32),
            pltpu.VMEM((BI,), jnp.int32),
            pltpu.VMEM((BI,), jnp.int32),
            pltpu.VMEM((BI, PW), jnp.float32),
            pltpu.VMEM((BI, PW), jnp.float32),
            pltpu.VMEM((BI, PW), jnp.float32),
            pltpu.VMEM((BI, PW), jnp.float32),
            pltpu.VMEM((BI, PW), jnp.float32),
            pltpu.VMEM((LANES,), jnp.int32),
            pltpu.SemaphoreType.DMA,
            pltpu.SemaphoreType.DMA,
            pltpu.SemaphoreType.DMA,
        ],
    )
    return kfn(pos_xy8, srcp, dstp)


def _sc_layer_body(h_hbm, sdl_hbm, prl_hbm, tot_hbm,
                   w_hbm, b_hbm, out_hbm,
                   agg_sh, src0, dstl0, prec0, xrows0,
                   src1, dstl1, prec1, xrows1, msg_v,
                   w_v, b_v, tv, sem0, sem1):
    c = lax.axis_index("c")
    s = lax.axis_index("s")
    iota = lax.iota(jnp.int32, LANES)
    pltpu.sync_copy(w_hbm, w_v)
    pltpu.sync_copy(b_hbm, b_v)
    wa = [w_v[pl.ds(16 * k, 16)] for k in range(4)]
    wb = [w_v[pl.ds(H + 16 * k, 16)] for k in range(4)]
    bb = [b_v[pl.ds(16 * k, 16)] for k in range(4)]
    z0 = jnp.full((LANES,), 0, jnp.int32)
    o1 = jnp.full((LANES,), 1, jnp.int32)
    bufs = ((src0, dstl0, prec0, xrows0, sem0),
            (src1, dstl1, prec1, xrows1, sem1))

    for p in range(NPASS):
        r = NPASS * c + p

        def zloop(e, carry):
            for k in range(4):
                msg_v[e, pl.ds(16 * k, 16)] = jnp.zeros((16,), jnp.float32)
            return carry
        lax.fori_loop(0, BLK, zloop, 0)
        for q in range(ROWS_PT // BLK):
            pltpu.sync_copy(msg_v, agg_sh.at[pl.ds(s * ROWS_PT + q * BLK, BLK)])
        plsc.subcore_barrier()

        for cc in range(NCORES):
            flatbase = ((cc * NSUB + s) * NRANGE + r) * CAP
            pltpu.sync_copy(tot_hbm.at[cc, s, r], tv)
            tvec = tv[...]
            total = tvec[0]
            trips = lax.shift_right_logical(total + (BLK - 1), BSH)
            bi_max = jnp.maximum(trips - 1, 0)

            def prep(bi_real, bset, flatbase=flatbase, bi_max=bi_max,
                     tvec=tvec):
                sv, dv, pv, xv, sg = bset
                o = flatbase + jnp.minimum(bi_real, bi_max) * BLK
                pltpu.sync_copy(sdl_hbm.at[pl.ds(o, BLK)], sv)
                pltpu.async_copy(prl_hbm.at[pl.ds(o, BLK), :], pv, sg)

                def uloop(g, uc):
                    gb = g * LANES
                    sd = sv[pl.ds(gb, LANES)]
                    lane = bi_real * BLK + gb + iota
                    m = lane < tvec
                    sdd = (lax.shift_left(gb + iota, RSH)
                           | (RNG + (g % 64) * 16 + iota))
                    sd = jnp.where(m, sd, sdd)
                    sv[pl.ds(gb, LANES)] = lax.shift_right_logical(sd, RSH)
                    dv[pl.ds(gb, LANES)] = sd & (RPAD - 1)
                    return uc
                lax.fori_loop(0, NGRP, uloop, 0)
                pltpu.async_copy(h_hbm.at[sv], xv, sg)

            def step(bi_real, bset, nbi_real, nbset, prefetch,
                     flatbase=flatbase):
                sv, dv, pv, xv, sg = bset
                pltpu.make_async_copy(
                    prl_hbm.at[pl.ds(flatbase, BLK), :], pv, sg).wait()
                pltpu.make_async_copy(h_hbm.at[sv], xv, sg).wait()
                if prefetch:
                    prep(nbi_real, nbset)

                def gloop(g, gc):
                    gb = g * LANES
                    for j in range(LANES):
                        e = gb + j
                        prow = pv[e, :]
                        us = jnp.take_along_axis(prow, z0, axis=0,
                                                 mode="promise_in_bounds")
                        vs = jnp.take_along_axis(prow, o1, axis=0,
                                                 mode="promise_in_bounds")
                        xr = xv[e, :]
                        for k in range(4):
                            t = us * wa[k] + vs * wb[k] + bb[k]
                            t = jnp.maximum(t, 0.0)
                            msg_v[e, pl.ds(16 * k, 16)] = t * xr
                    return gc
                lax.fori_loop(0, NGRP, gloop, 0)
                pltpu.sync_copy(msg_v, agg_sh.at[dv], add=True)

            prep(0, bufs[0])

            def pairloop(i, carry):
                b0 = i * 2
                step(b0, bufs[0], b0 + 1, bufs[1], True)
                step(b0 + 1, bufs[1], b0 + 2, bufs[0], True)
                return carry
            lax.fori_loop(0, lax.shift_right_logical(trips + 1, 1),
                          pairloop, 0)
            pltpu.make_async_copy(
                prl_hbm.at[pl.ds(flatbase, BLK), :], prec0, sem0).wait()
            pltpu.make_async_copy(h_hbm.at[src0], xrows0, sem0).wait()
        plsc.subcore_barrier()

        pltpu.sync_copy(agg_sh.at[pl.ds(s * ROWS_PT, ROWS_PT)],
                        out_hbm.at[r, pl.ds(s * ROWS_PT, ROWS_PT)])
        plsc.subcore_barrier()


def _sc_layer(h, edge_lists, w_flat, b_in):
    sdl, prl, tot = edge_lists
    mesh = plsc.VectorSubcoreMesh(core_axis_name="c", subcore_axis_name="s",
                                  num_cores=NCORES, num_subcores=NSUB)
    kfn = pl.kernel(
        _sc_layer_body,
        out_type=jax.ShapeDtypeStruct((NRANGE, RPAD, H), jnp.float32),
        mesh=mesh,
        compiler_params=pltpu.CompilerParams(use_tc_tiling_on_sc=False),
        scratch_types=[
            pltpu.VMEM_SHARED((RPAD, H), jnp.float32),
            pltpu.VMEM((BLK,), jnp.int32),
            pltpu.VMEM((BLK,), jnp.int32),
            pltpu.VMEM((BLK, PW), jnp.float32),
            pltpu.VMEM((BLK, D), jnp.float32),
            pltpu.VMEM((BLK,), jnp.int32),
            pltpu.VMEM((BLK,), jnp.int32),
            pltpu.VMEM((BLK, PW), jnp.float32),
            pltpu.VMEM((BLK, D), jnp.float32),
            pltpu.VMEM((BLK, H), jnp.float32),
            pltpu.VMEM((2 * H,), jnp.float32),
            pltpu.VMEM((H,), jnp.float32),
            pltpu.VMEM((LANES,), jnp.int32),
            pltpu.SemaphoreType.DMA,
            pltpu.SemaphoreType.DMA,
        ],
    )
    return kfn(h, sdl, prl, tot, w_flat, b_in)


def _tc_affine(agg2, w_out, b_out2):
    AB = 512

    def body(agg_ref, w_ref, b_ref, out_ref):
        out_ref[...] = (
            jnp.dot(agg_ref[...], w_ref[...],
                    preferred_element_type=jnp.float32,
                    precision=lax.Precision.HIGHEST)
            + b_ref[...])

    return pl.pallas_call(
        body,
        grid=(NPAD // AB,),
        in_specs=[
            pl.BlockSpec((AB, H), lambda i: (i, 0)),
            pl.BlockSpec((H, D), lambda i: (0, 0)),
            pl.BlockSpec((1, D), lambda i: (0, 0)),
        ],
        out_specs=pl.BlockSpec((AB, D), lambda i: (i, 0)),
        out_shape=jax.ShapeDtypeStruct((NPAD, D), jnp.float32),
    )(agg2, w_out, b_out2)


def _tc_head(h, batch3, w_lin, b_lin2):
    BN = 1000
    NB = N // BN

    def body(h_ref, b_ref, wl_ref, bl_ref, out_ref, acc, cnt):
        i = pl.program_id(0)

        @pl.when(i == 0)
        def _init():
            acc[...] = jnp.zeros_like(acc)
            cnt[...] = jnp.zeros_like(cnt)

        gi = lax.broadcasted_iota(jnp.int32, (G, BN), 0)
        oht = (b_ref[0] == gi).astype(jnp.float32)
        acc[...] += lax.dot_general(oht, h_ref[...],
                                    (((1,), (0,)), ((), ())),
                                    preferred_element_type=jnp.float32,
                                    precision=lax.Precision.HIGHEST)
        cnt[...] += jnp.sum(oht, axis=1, keepdims=True)

        @pl.when(i == NB - 1)
        def _fin():
            pooled = acc[...] / jnp.maximum(cnt[...], 1.0)
            logits = (jnp.dot(pooled, wl_ref[...],
                              preferred_element_type=jnp.float32,
                              precision=lax.Precision.HIGHEST)
                      + bl_ref[...])
            mx = jnp.max(logits, axis=0, keepdims=True)
            z = logits - mx
            lse = jnp.log(jnp.sum(jnp.exp(z), axis=0, keepdims=True))
            out_ref[...] = z - lse

    return pl.pallas_call(
        body,
        grid=(NB,),
        in_specs=[
            pl.BlockSpec((BN, D), lambda i: (i, 0)),
            pl.BlockSpec((1, 1, BN), lambda i: (i, 0, 0)),
            pl.BlockSpec((D, NCLS), lambda i: (0, 0)),
            pl.BlockSpec((1, NCLS), lambda i: (0, 0)),
        ],
        out_specs=pl.BlockSpec((G, NCLS), lambda i: (0, 0)),
        out_shape=jax.ShapeDtypeStruct((G, NCLS), jnp.float32),
        scratch_shapes=[pltpu.VMEM((G, D), jnp.float32),
                        pltpu.VMEM((G, 1), jnp.float32)],
    )(h, batch3, w_lin, b_lin2)


def kernel(x, pos, edge_index, batch,
           W_in1, b_in1, W_out1, b_out1,
           W_in2, b_in2, W_out2, b_out2,
           W_in3, b_in3, W_out3, b_out3,
           W_lin, b_lin):
    srcp = edge_index[0] + (edge_index[0] // RNG) * (RPAD - RNG)
    dstp = edge_index[1] + (edge_index[1] // RNG) * (RPAD - RNG)
    pos_pad = jnp.pad(pos.reshape(NRANGE, RNG, 2),
                      ((0, 0), (0, RPAD - RNG), (0, PW - 2)))
    pos_xy8 = pos_pad.reshape(NPAD, PW)
    x_pad = jnp.pad(x.reshape(NRANGE, RNG, D),
                    ((0, 0), (0, RPAD - RNG), (0, 0))).reshape(NPAD, D)
    batch3 = batch.reshape(N // 1000, 1, 1000)

    edge_lists = _bin_edges(pos_xy8, srcp, dstp)

    hp = x_pad
    for (wi, bi, wo, bo) in ((W_in1, b_in1, W_out1, b_out1),
                             (W_in2, b_in2, W_out2, b_out2),
                             (W_in3, b_in3, W_out3, b_out3)):
        agg = _sc_layer(hp, edge_lists, wi.reshape(2 * H), bi)
        hp = _tc_affine(agg.reshape(NPAD, H), wo, bo.reshape(1, D))
    h = hp.reshape(NRANGE, RPAD, D)[:, :RNG, :].reshape(N, D)
    return _tc_head(h, batch3, W_lin, b_lin.reshape(1, NCLS))
